# Initial kernel scaffold; baseline (speedup 1.0000x reference)
import jax
import jax.numpy as jnp
from jax.experimental import pallas as pl


def kernel(embedding, W):
    raise NotImplementedError("write your pallas kernel here")


# speedup vs baseline: 2.0531x; 2.0531x over previous
import jax
import jax.numpy as jnp
from jax.experimental import pallas as pl

_B_TILE = 1024
_K = 12


def _tc_body(x_ref, w_ref, o_ref):
    x = x_ref[...]
    w = w_ref[...]
    s = jax.lax.dot_general(w, x, (((1,), (1,)), ((), ())),
                            preferred_element_type=jnp.float32)
    cur = jnp.abs(s)
    tb = cur.shape[1]
    acc = jnp.zeros((1, tb), jnp.float32)
    rem = jnp.full((1, tb), float(_K), jnp.float32)
    for _ in range(_K):
        m = jnp.max(cur, axis=0, keepdims=True)
        eq = cur == m
        c = jnp.sum(jnp.where(eq, 1.0, 0.0), axis=0, keepdims=True)
        take = jnp.minimum(c, rem)
        acc = acc + take * m
        rem = rem - take
        cur = jnp.where(eq, -1.0, cur)
    o_ref[...] = acc * (1.0 / _K)


def kernel(embedding, W):
    B, emb = embedding.shape
    out = pl.pallas_call(
        _tc_body,
        grid=(B // _B_TILE,),
        in_specs=[
            pl.BlockSpec((_B_TILE, emb), lambda i: (i, 0)),
            pl.BlockSpec(W.shape, lambda i: (0, 0)),
        ],
        out_specs=pl.BlockSpec((1, _B_TILE), lambda i: (0, i)),
        out_shape=jax.ShapeDtypeStruct((1, B), jnp.float32),
    )(embedding, W)
    return out.reshape(B, 1)


# speedup vs baseline: 2.2212x; 1.0819x over previous
import jax
import jax.numpy as jnp
from jax.experimental import pallas as pl

_B_TILE = 1024
_K = 12


def _tc_body(x_ref, w_ref, o_ref):
    x = x_ref[...]
    w = w_ref[...]
    s = jax.lax.dot_general(w, x, (((1,), (1,)), ((), ())),
                            preferred_element_type=jnp.float32)
    a = jnp.abs(s)
    tb = a.shape[1]
    bits = jax.lax.bitcast_convert_type(a, jnp.int32)
    sub = jax.lax.broadcasted_iota(jnp.int32, a.shape, 0)
    cur = jnp.bitwise_or(jnp.bitwise_and(bits, ~jnp.int32(31)), sub)
    acc = jnp.zeros((1, tb), jnp.float32)
    for _ in range(_K):
        m = jnp.max(cur, axis=0, keepdims=True)
        acc = acc + jax.lax.bitcast_convert_type(m, jnp.float32)
        cur = jnp.where(cur == m, jnp.int32(-1), cur)
    o_ref[...] = acc * (1.0 / _K)


def kernel(embedding, W):
    B, emb = embedding.shape
    out = pl.pallas_call(
        _tc_body,
        grid=(B // _B_TILE,),
        in_specs=[
            pl.BlockSpec((_B_TILE, emb), lambda i: (i, 0)),
            pl.BlockSpec(W.shape, lambda i: (0, 0)),
        ],
        out_specs=pl.BlockSpec((1, _B_TILE), lambda i: (0, i)),
        out_shape=jax.ShapeDtypeStruct((1, B), jnp.float32),
    )(embedding, W)
    return out.reshape(B, 1)


# speedup vs baseline: 3.5066x; 1.5787x over previous
import jax
import jax.numpy as jnp
from jax.experimental import pallas as pl

_B_TILE = 2048
_K = 12


def _tc_body(x_ref, w_ref, o_ref):
    x = x_ref[...]
    w = w_ref[...]
    s = jax.lax.dot_general(w, x, (((1,), (1,)), ((), ())),
                            preferred_element_type=jnp.float32)
    a = jnp.abs(s)
    tb = a.shape[1]
    bits = jax.lax.bitcast_convert_type(a, jnp.int32)
    sub = jax.lax.broadcasted_iota(jnp.int32, a.shape, 0)
    cur = jax.lax.bitcast_convert_type(
        jnp.bitwise_or(jnp.bitwise_and(bits, ~jnp.int32(31)), sub),
        jnp.float32)
    acc = jnp.zeros((1, tb), jnp.float32)
    for _ in range(_K):
        m = jnp.max(cur, axis=0, keepdims=True)
        acc = acc + jnp.maximum(m, 0.0)
        cur = jnp.where(cur == m, -1.0, cur)
    o_ref[...] = acc * (1.0 / _K)


def kernel(embedding, W):
    B, emb = embedding.shape
    out = pl.pallas_call(
        _tc_body,
        grid=(B // _B_TILE,),
        in_specs=[
            pl.BlockSpec((_B_TILE, emb), lambda i: (i, 0)),
            pl.BlockSpec(W.shape, lambda i: (0, 0)),
        ],
        out_specs=pl.BlockSpec((1, _B_TILE), lambda i: (0, i)),
        out_shape=jax.ShapeDtypeStruct((1, B), jnp.float32),
    )(embedding, W)
    return out.reshape(B, 1)


# speedup vs baseline: 4.7397x; 1.3516x over previous
import jax
import jax.numpy as jnp
from jax.experimental import pallas as pl

_B_TILE = 4096
_K = 12


def _tc_body(x_ref, w_ref, o_ref):
    x = x_ref[...]
    w = w_ref[...]
    s = jax.lax.dot_general(w, x, (((1,), (1,)), ((), ())),
                            preferred_element_type=jnp.float32)
    a = jnp.abs(s)
    tb = a.shape[1]
    bits = jax.lax.bitcast_convert_type(a, jnp.int32)
    sub = jax.lax.broadcasted_iota(jnp.int32, a.shape, 0)
    cur = jax.lax.bitcast_convert_type(
        jnp.bitwise_or(jnp.bitwise_and(bits, ~jnp.int32(31)), sub),
        jnp.float32)
    acc = jnp.zeros((1, tb), jnp.float32)
    for _ in range(_K):
        m = jnp.max(cur, axis=0, keepdims=True)
        acc = acc + jnp.maximum(m, 0.0)
        cur = jnp.where(cur == m, -1.0, cur)
    o_ref[...] = acc * (1.0 / _K)


def kernel(embedding, W):
    B, emb = embedding.shape
    out = pl.pallas_call(
        _tc_body,
        grid=(B // _B_TILE,),
        in_specs=[
            pl.BlockSpec((_B_TILE, emb), lambda i: (i, 0)),
            pl.BlockSpec(W.shape, lambda i: (0, 0)),
        ],
        out_specs=pl.BlockSpec((1, _B_TILE), lambda i: (0, i)),
        out_shape=jax.ShapeDtypeStruct((1, B), jnp.float32),
    )(embedding, W)
    return out.reshape(B, 1)


# speedup vs baseline: 5.4582x; 1.1516x over previous
import jax
import jax.numpy as jnp
from jax.experimental import pallas as pl

_B_TILE = 8192
_K = 12


def _tc_body(x_ref, w_ref, o_ref):
    x = x_ref[...]
    w = w_ref[...]
    s = jax.lax.dot_general(w, x, (((1,), (1,)), ((), ())),
                            preferred_element_type=jnp.float32)
    a = jnp.abs(s)
    tb = a.shape[1]
    bits = jax.lax.bitcast_convert_type(a, jnp.int32)
    sub = jax.lax.broadcasted_iota(jnp.int32, a.shape, 0)
    cur = jax.lax.bitcast_convert_type(
        jnp.bitwise_or(jnp.bitwise_and(bits, ~jnp.int32(31)), sub),
        jnp.float32)
    acc = jnp.zeros((1, tb), jnp.float32)
    for _ in range(_K):
        m = jnp.max(cur, axis=0, keepdims=True)
        acc = acc + jnp.maximum(m, 0.0)
        cur = jnp.where(cur == m, -1.0, cur)
    o_ref[...] = acc * (1.0 / _K)


def kernel(embedding, W):
    B, emb = embedding.shape
    out = pl.pallas_call(
        _tc_body,
        grid=(B // _B_TILE,),
        in_specs=[
            pl.BlockSpec((_B_TILE, emb), lambda i: (i, 0)),
            pl.BlockSpec(W.shape, lambda i: (0, 0)),
        ],
        out_specs=pl.BlockSpec((1, _B_TILE), lambda i: (0, i)),
        out_shape=jax.ShapeDtypeStruct((1, B), jnp.float32),
    )(embedding, W)
    return out.reshape(B, 1)


# speedup vs baseline: 5.4979x; 1.0073x over previous
import jax
import jax.numpy as jnp
from jax.experimental import pallas as pl

_B_TILE = 16384
_K = 12


def _tc_body(x_ref, w_ref, o_ref):
    x = x_ref[...]
    w = w_ref[...]
    s = jax.lax.dot_general(w, x, (((1,), (1,)), ((), ())),
                            preferred_element_type=jnp.float32)
    a = jnp.abs(s)
    tb = a.shape[1]
    bits = jax.lax.bitcast_convert_type(a, jnp.int32)
    sub = jax.lax.broadcasted_iota(jnp.int32, a.shape, 0)
    cur = jax.lax.bitcast_convert_type(
        jnp.bitwise_or(jnp.bitwise_and(bits, ~jnp.int32(31)), sub),
        jnp.float32)
    acc = jnp.zeros((1, tb), jnp.float32)
    for _ in range(_K):
        m = jnp.max(cur, axis=0, keepdims=True)
        acc = acc + jnp.maximum(m, 0.0)
        cur = jnp.where(cur == m, -1.0, cur)
    o_ref[...] = acc * (1.0 / _K)


def kernel(embedding, W):
    B, emb = embedding.shape
    out = pl.pallas_call(
        _tc_body,
        grid=(B // _B_TILE,),
        in_specs=[
            pl.BlockSpec((_B_TILE, emb), lambda i: (i, 0)),
            pl.BlockSpec(W.shape, lambda i: (0, 0)),
        ],
        out_specs=pl.BlockSpec((1, _B_TILE), lambda i: (0, i)),
        out_shape=jax.ShapeDtypeStruct((1, B), jnp.float32),
    )(embedding, W)
    return out.reshape(B, 1)
